# Initial kernel scaffold; baseline (speedup 1.0000x reference)
import functools

import jax
import jax.numpy as jnp
from jax import lax
from jax.experimental import pallas as pl
from jax.experimental.pallas import tpu as pltpu
from jax.experimental.pallas import tpu_sc as plsc

N = 10000
E = 320000
D = 128
EPS = 1e-09

NC = 2
NS = 16
K = 128
E_PER_TILE = -(-E // (NC * NS * K)) * K
E_PAD = E_PER_TILE * NC * NS
CHUNKS = E_PER_TILE // K
ROWS_PER_TILE = N // NS

BN = 400
GRID = N // BN


def _tc_a_body(x_ref, w0_ref, w1_ref, b0_ref, att0_ref, att1_ref, attb_ref,
               sc0_ref, off0_ref, rs_ref, vwn_ref, att_ref):
    x = x_ref[:]
    h0 = jnp.dot(x, w0_ref[:], preferred_element_type=jnp.float32)
    h0 = jnp.maximum(h0 + b0_ref[:], 0.0)
    m = jnp.mean(h0, axis=1, keepdims=True)
    v = jnp.mean((h0 - m) ** 2, axis=1, keepdims=True)
    rs_ref[:] = sc0_ref[:] * (h0 - m) * lax.rsqrt(v + EPS) + off0_ref[:]
    h1 = jnp.dot(x, w1_ref[:], preferred_element_type=jnp.float32)
    vwn_ref[:] = h1
    a_n = lax.dot_general(att1_ref[:], h1, (((1,), (1,)), ((), ())),
                          preferred_element_type=jnp.float32) + attb_ref[0, 1]
    a_s = lax.dot_general(att0_ref[:], h1, (((1,), (1,)), ((), ())),
                          preferred_element_type=jnp.float32) + attb_ref[0, 0]
    att_ref[0] = jnp.concatenate([a_n, a_s], axis=0)


def _tc_a(vecs, w0, w1, b0, att0, att1, attb, sc0, off0):
    full = lambda i: (0, 0)
    return pl.pallas_call(
        _tc_a_body,
        grid=(GRID,),
        in_specs=[
            pl.BlockSpec((BN, D), lambda i: (i, 0)),
            pl.BlockSpec((D, D), full),
            pl.BlockSpec((D, D), full),
            pl.BlockSpec((1, D), full),
            pl.BlockSpec((1, D), full),
            pl.BlockSpec((1, D), full),
            pl.BlockSpec(memory_space=pltpu.SMEM),
            pl.BlockSpec((1, D), full),
            pl.BlockSpec((1, D), full),
        ],
        out_specs=[
            pl.BlockSpec((BN, D), lambda i: (i, 0)),
            pl.BlockSpec((BN, D), lambda i: (i, 0)),
            pl.BlockSpec((1, 2, BN), lambda i: (i, 0, 0)),
        ],
        out_shape=[
            jax.ShapeDtypeStruct((N, D), jnp.float32),
            jax.ShapeDtypeStruct((N, D), jnp.float32),
            jax.ShapeDtypeStruct((GRID, 2, BN), jnp.float32),
        ],
    )(vecs, w0, w1, b0, att0, att1, attb, sc0, off0)


def _sc_body(row2_hbm, col2_hbm, vals2_hbm, an_hbm, as_hbm, vwn_hbm,
             zeros_hbm, out_hbm,
             an_v, as_v, row_v, col_v, vals_v, w_v, rows_v, agg_sh):
    c = lax.axis_index("c")
    s = lax.axis_index("s")

    pltpu.sync_copy(an_hbm, an_v)
    pltpu.sync_copy(as_hbm, as_v)

    pltpu.sync_copy(zeros_hbm, agg_sh.at[pl.ds(s * ROWS_PER_TILE, ROWS_PER_TILE)])
    plsc.subcore_barrier()

    chunk0 = (c * NS + s) * CHUNKS

    def chunk_body(k, carry):
        ck = chunk0 + k
        pltpu.sync_copy(row2_hbm.at[ck], row_v)
        pltpu.sync_copy(col2_hbm.at[ck], col_v)
        pltpu.sync_copy(vals2_hbm.at[ck], vals_v)

        for i in range(K // 16):
            sl = pl.ds(i * 16, 16)
            a1 = plsc.load_gather(an_v, [col_v[sl]])
            a2 = plsc.load_gather(as_v, [row_v[sl]])
            w_v[sl] = vals_v[sl] * jnp.maximum(a1 + a2, 0.0)

        pltpu.sync_copy(vwn_hbm.at[col_v], rows_v)

        def mul_body(e, _):
            we = w_v[e]
            for j in range(D // 16):
                rsl = pl.ds(j * 16, 16)
                rows_v[e, rsl] = rows_v[e, rsl] * we
            return 0

        lax.fori_loop(0, K, mul_body, 0)

        pltpu.sync_copy(rows_v, agg_sh.at[row_v], add=True)
        return carry

    lax.fori_loop(0, CHUNKS, chunk_body, 0)

    plsc.subcore_barrier()
    pltpu.sync_copy(agg_sh.at[pl.ds(s * ROWS_PER_TILE, ROWS_PER_TILE)],
                    out_hbm.at[c, pl.ds(s * ROWS_PER_TILE, ROWS_PER_TILE)])


def _sc_aggregate(row2, col2, vals2, a_n, a_s, vwn, zeros_rows):
    mesh = plsc.VectorSubcoreMesh(core_axis_name="c", subcore_axis_name="s")
    f = pl.kernel(
        _sc_body,
        out_type=jax.ShapeDtypeStruct((NC, N, D), jnp.float32),
        mesh=mesh,
        scratch_types=[
            pltpu.VMEM((N,), jnp.float32),
            pltpu.VMEM((N,), jnp.float32),
            pltpu.VMEM((K,), jnp.int32),
            pltpu.VMEM((K,), jnp.int32),
            pltpu.VMEM((K,), jnp.float32),
            pltpu.VMEM((K,), jnp.float32),
            pltpu.VMEM((K, D), jnp.float32),
            pltpu.VMEM_SHARED((N, D), jnp.float32),
        ],
    )
    return f(row2, col2, vals2, a_n, a_s, vwn, zeros_rows)


def _tc_c_body(p_ref, rs_ref, b1_ref, sc1_ref, off1_ref, out_ref):
    agg = p_ref[0] + p_ref[1]
    rn = jnp.maximum(agg, 0.0) + b1_ref[:]
    m = jnp.mean(rn, axis=1, keepdims=True)
    v = jnp.mean((rn - m) ** 2, axis=1, keepdims=True)
    out_ref[:] = (sc1_ref[:] * (rn - m) * lax.rsqrt(v + EPS) + off1_ref[:]
                  + rs_ref[:])


def _tc_c(partials, ret_self, b1, sc1, off1):
    full = lambda i: (0, 0)
    return pl.pallas_call(
        _tc_c_body,
        grid=(GRID,),
        in_specs=[
            pl.BlockSpec((NC, BN, D), lambda i: (0, i, 0)),
            pl.BlockSpec((BN, D), lambda i: (i, 0)),
            pl.BlockSpec((1, D), full),
            pl.BlockSpec((1, D), full),
            pl.BlockSpec((1, D), full),
        ],
        out_specs=pl.BlockSpec((BN, D), lambda i: (i, 0)),
        out_shape=jax.ShapeDtypeStruct((N, D), jnp.float32),
    )(partials, ret_self, b1, sc1, off1)


def kernel(vecs, edge_index, edge_vals, W0, b0, W1, b1, att0, att1,
           att_b0, att_b1, off0, sc0, off1, sc1):
    b0r = b0.reshape(1, D)
    b1r = b1.reshape(1, D)
    attb = jnp.concatenate([att_b0, att_b1]).reshape(1, 2)

    ret_self, vwn, att = _tc_a(vecs, W0, W1, b0r, att0, att1, attb, sc0, off0)
    a_n = att[:, 0, :].reshape(N)
    a_s = att[:, 1, :].reshape(N)

    pad = E_PAD - E
    row = jnp.concatenate([edge_index[0], jnp.zeros((pad,), jnp.int32)])
    col = jnp.concatenate([edge_index[1], jnp.zeros((pad,), jnp.int32)])
    vals = jnp.concatenate([edge_vals, jnp.zeros((pad,), jnp.float32)])
    row2 = row.reshape(E_PAD // K, K)
    col2 = col.reshape(E_PAD // K, K)
    vals2 = vals.reshape(E_PAD // K, K)
    zeros_rows = jnp.zeros((ROWS_PER_TILE, D), jnp.float32)

    partials = _sc_aggregate(row2, col2, vals2, a_n, a_s, vwn, zeros_rows)

    return _tc_c(partials, ret_self, b1r, sc1, off1)


# speedup vs baseline: 11.0695x; 11.0695x over previous
import functools

import jax
import jax.numpy as jnp
from jax import lax
from jax.experimental import pallas as pl
from jax.experimental.pallas import tpu as pltpu
from jax.experimental.pallas import tpu_sc as plsc

N = 10000
E = 320000
D = 128
EPS = 1e-09

NC = 2
NS = 16
K = 128
E_PER_TILE = -(-E // (NC * NS * K)) * K
E_PAD = E_PER_TILE * NC * NS
CHUNKS = E_PER_TILE // K
ROWS_PER_TILE = 632
NPAD = ROWS_PER_TILE * NS

BN = 400
GRID = N // BN


def _tc_a_body(x_ref, w0_ref, w1_ref, b0_ref, att0_ref, att1_ref, attb_ref,
               sc0_ref, off0_ref, rs_ref, vwn_ref, att_ref):
    x = x_ref[:]
    h0 = jnp.dot(x, w0_ref[:], preferred_element_type=jnp.float32)
    h0 = jnp.maximum(h0 + b0_ref[:], 0.0)
    m = jnp.mean(h0, axis=1, keepdims=True)
    v = jnp.mean((h0 - m) ** 2, axis=1, keepdims=True)
    rs_ref[:] = sc0_ref[:] * (h0 - m) * lax.rsqrt(v + EPS) + off0_ref[:]
    h1 = jnp.dot(x, w1_ref[:], preferred_element_type=jnp.float32)
    vwn_ref[:] = h1
    a_n = lax.dot_general(att1_ref[:], h1, (((1,), (1,)), ((), ())),
                          preferred_element_type=jnp.float32) + attb_ref[0, 1]
    a_s = lax.dot_general(att0_ref[:], h1, (((1,), (1,)), ((), ())),
                          preferred_element_type=jnp.float32) + attb_ref[0, 0]
    att_ref[0] = jnp.concatenate([a_n, a_s], axis=0)


def _tc_a(vecs, w0, w1, b0, att0, att1, attb, sc0, off0):
    full = lambda i: (0, 0)
    return pl.pallas_call(
        _tc_a_body,
        grid=(GRID,),
        in_specs=[
            pl.BlockSpec((BN, D), lambda i: (i, 0)),
            pl.BlockSpec((D, D), full),
            pl.BlockSpec((D, D), full),
            pl.BlockSpec((1, D), full),
            pl.BlockSpec((1, D), full),
            pl.BlockSpec((1, D), full),
            pl.BlockSpec(memory_space=pltpu.SMEM),
            pl.BlockSpec((1, D), full),
            pl.BlockSpec((1, D), full),
        ],
        out_specs=[
            pl.BlockSpec((BN, D), lambda i: (i, 0)),
            pl.BlockSpec((BN, D), lambda i: (i, 0)),
            pl.BlockSpec((1, 2, BN), lambda i: (i, 0, 0)),
        ],
        out_shape=[
            jax.ShapeDtypeStruct((N, D), jnp.float32),
            jax.ShapeDtypeStruct((N, D), jnp.float32),
            jax.ShapeDtypeStruct((GRID, 2, BN), jnp.float32),
        ],
    )(vecs, w0, w1, b0, att0, att1, attb, sc0, off0)


def _sc_body(row2_hbm, col2_hbm, vals2_hbm, an_hbm, as_hbm, vwn_hbm,
             zeros_hbm, out_hbm,
             an_v, as_v, row_v, col_v, vals_v, w_v, rows_v, agg_sh):
    c = lax.axis_index("c")
    s = lax.axis_index("s")

    pltpu.sync_copy(an_hbm, an_v)
    pltpu.sync_copy(as_hbm, as_v)

    pltpu.sync_copy(zeros_hbm, agg_sh.at[pl.ds(s * ROWS_PER_TILE, ROWS_PER_TILE)])
    plsc.subcore_barrier()

    chunk0 = (c * NS + s) * CHUNKS

    def chunk_body(k, carry):
        ck = chunk0 + k
        pltpu.sync_copy(row2_hbm.at[ck], row_v)
        pltpu.sync_copy(col2_hbm.at[ck], col_v)
        pltpu.sync_copy(vals2_hbm.at[ck], vals_v)

        for i in range(K // 16):
            sl = pl.ds(i * 16, 16)
            a1 = plsc.load_gather(an_v, [col_v[sl]])
            a2 = plsc.load_gather(as_v, [row_v[sl]])
            w_v[sl] = vals_v[sl] * jnp.maximum(a1 + a2, 0.0)

        pltpu.sync_copy(vwn_hbm.at[col_v], rows_v)

        def mul_body(i, _):
            base = i * 16
            w16 = w_v[pl.ds(base, 16)]
            for l in range(16):
                we = w16[l]
                for j in range(D // 16):
                    rsl = pl.ds(j * 16, 16)
                    rows_v[base + l, rsl] = rows_v[base + l, rsl] * we
            return 0

        lax.fori_loop(0, K // 16, mul_body, 0)

        pltpu.sync_copy(rows_v, agg_sh.at[row_v], add=True)
        return carry

    lax.fori_loop(0, CHUNKS, chunk_body, 0)

    plsc.subcore_barrier()
    pltpu.sync_copy(agg_sh.at[pl.ds(s * ROWS_PER_TILE, ROWS_PER_TILE)],
                    out_hbm.at[c, pl.ds(s * ROWS_PER_TILE, ROWS_PER_TILE)])


def _sc_aggregate(row2, col2, vals2, a_n, a_s, vwn, zeros_rows):
    mesh = plsc.VectorSubcoreMesh(core_axis_name="c", subcore_axis_name="s")
    f = pl.kernel(
        _sc_body,
        out_type=jax.ShapeDtypeStruct((NC, NPAD, D), jnp.float32),
        mesh=mesh,
        scratch_types=[
            pltpu.VMEM((N,), jnp.float32),
            pltpu.VMEM((N,), jnp.float32),
            pltpu.VMEM((K,), jnp.int32),
            pltpu.VMEM((K,), jnp.int32),
            pltpu.VMEM((K,), jnp.float32),
            pltpu.VMEM((K,), jnp.float32),
            pltpu.VMEM((K, D), jnp.float32),
            pltpu.VMEM_SHARED((NPAD, D), jnp.float32),
        ],
        compiler_params=pltpu.CompilerParams(needs_layout_passes=False),
    )
    return f(row2, col2, vals2, a_n, a_s, vwn, zeros_rows)


def _tc_c_body(p_ref, rs_ref, b1_ref, sc1_ref, off1_ref, out_ref):
    agg = p_ref[0] + p_ref[1]
    rn = jnp.maximum(agg, 0.0) + b1_ref[:]
    m = jnp.mean(rn, axis=1, keepdims=True)
    v = jnp.mean((rn - m) ** 2, axis=1, keepdims=True)
    out_ref[:] = (sc1_ref[:] * (rn - m) * lax.rsqrt(v + EPS) + off1_ref[:]
                  + rs_ref[:])


def _tc_c(partials, ret_self, b1, sc1, off1):
    full = lambda i: (0, 0)
    return pl.pallas_call(
        _tc_c_body,
        grid=(GRID,),
        in_specs=[
            pl.BlockSpec((NC, BN, D), lambda i: (0, i, 0)),
            pl.BlockSpec((BN, D), lambda i: (i, 0)),
            pl.BlockSpec((1, D), full),
            pl.BlockSpec((1, D), full),
            pl.BlockSpec((1, D), full),
        ],
        out_specs=pl.BlockSpec((BN, D), lambda i: (i, 0)),
        out_shape=jax.ShapeDtypeStruct((N, D), jnp.float32),
    )(partials, ret_self, b1, sc1, off1)


def kernel(vecs, edge_index, edge_vals, W0, b0, W1, b1, att0, att1,
           att_b0, att_b1, off0, sc0, off1, sc1):
    b0r = b0.reshape(1, D)
    b1r = b1.reshape(1, D)
    attb = jnp.concatenate([att_b0, att_b1]).reshape(1, 2)

    ret_self, vwn, att = _tc_a(vecs, W0, W1, b0r, att0, att1, attb, sc0, off0)
    a_n = att[:, 0, :].reshape(N)
    a_s = att[:, 1, :].reshape(N)

    pad = E_PAD - E
    row = jnp.concatenate([edge_index[0], jnp.zeros((pad,), jnp.int32)])
    col = jnp.concatenate([edge_index[1], jnp.zeros((pad,), jnp.int32)])
    vals = jnp.concatenate([edge_vals, jnp.zeros((pad,), jnp.float32)])
    row2 = row.reshape(E_PAD // K, K)
    col2 = col.reshape(E_PAD // K, K)
    vals2 = vals.reshape(E_PAD // K, K)
    zeros_rows = jnp.zeros((ROWS_PER_TILE, D), jnp.float32)

    partials = _sc_aggregate(row2, col2, vals2, a_n, a_s, vwn, zeros_rows)

    return _tc_c(partials, ret_self, b1r, sc1, off1)
